# Initial kernel scaffold; baseline (speedup 1.0000x reference)
import jax
import jax.numpy as jnp
from jax.experimental import pallas as pl


def kernel(x, w0, b0, w1, b1, w2, b2, w3, b3, w4, b4, w5, b5):
    raise NotImplementedError("write your pallas kernel here")


# speedup vs baseline: 7.5526x; 7.5526x over previous
import functools

import jax
import jax.numpy as jnp
from jax.experimental import pallas as pl
from jax.experimental.pallas import tpu as pltpu


def _finish(acc, b_ref, o_ref, act, bh, OW):
    acc = acc + b_ref[...]
    if act == "relu":
        acc = jnp.maximum(acc, 0.0)
    elif act == "tanh":
        acc = jnp.tanh(acc)
    o_ref[...] = acc.reshape(1, bh, OW, -1).astype(o_ref.dtype)


def _conv_s1_kernel(x_ref, w_ref, b_ref, o_ref, *, kH, kW, bh, OW, act):
    y0 = pl.program_id(1) * bh
    taps = []
    for ki in range(kH):
        rows = x_ref[0, pl.ds(y0 + ki, bh), :, :]
        for kj in range(kW):
            taps.append(rows[:, kj:kj + OW, :])
    cin = x_ref.shape[3]
    patches = jnp.concatenate(taps, axis=2).reshape(bh * OW, kH * kW * cin)
    acc = jnp.dot(patches, w_ref[...], preferred_element_type=jnp.float32)
    _finish(acc, b_ref, o_ref, act, bh, OW)


def _conv7_rgb_kernel(x_ref, w_ref, b_ref, o_ref, *, bh, OW, act):
    y0 = pl.program_id(1) * bh
    block = x_ref[0, pl.ds(y0, 2 * bh), :]
    taps = []
    for ki in range(7):
        taps.append(block[ki:ki + bh, :].reshape(bh, OW, 21))
    patches = jnp.concatenate(taps, axis=2).reshape(bh * OW, 147)
    acc = jnp.dot(patches, w_ref[...], preferred_element_type=jnp.float32)
    _finish(acc, b_ref, o_ref, act, bh, OW)


def _conv_s2_kernel(x00, x01, x10, x11, w_ref, b_ref, o_ref, *, bh, OW, act):
    planes = (x00, x01, x10, x11)
    y0 = pl.program_id(1) * bh
    taps = []
    for ki in range(3):
        for kj in range(3):
            ref = planes[(ki % 2) * 2 + (kj % 2)]
            rows = ref[0, pl.ds(y0 + ki // 2, bh), :, :]
            taps.append(rows[:, kj // 2:kj // 2 + OW, :])
    cin = x00.shape[3]
    patches = jnp.concatenate(taps, axis=2).reshape(bh * OW, 9 * cin)
    acc = jnp.dot(patches, w_ref[...], preferred_element_type=jnp.float32)
    _finish(acc, b_ref, o_ref, act, bh, OW)


_COMPILER = dict(
    dimension_semantics=("parallel", "arbitrary"),
    vmem_limit_bytes=100 * 1024 * 1024,
)


def _conv_s1(x, w, b, kH, kW, pad, act, out_dtype, bh):
    if pad != (0, 0):
        x = jnp.pad(x, ((0, 0), pad, pad, (0, 0)))
    N, Hp, Wp, Cin = x.shape
    K, Cout = w.shape
    OH, OW = Hp - kH + 1, Wp - kW + 1
    assert OH % bh == 0, (OH, bh)
    b2 = b.astype(jnp.float32).reshape(1, Cout)
    kern = functools.partial(_conv_s1_kernel, kH=kH, kW=kW, bh=bh, OW=OW,
                             act=act)
    return pl.pallas_call(
        kern,
        out_shape=jax.ShapeDtypeStruct((N, OH, OW, Cout), out_dtype),
        grid=(N, OH // bh),
        in_specs=[
            pl.BlockSpec((1, Hp, Wp, Cin), lambda n, r: (n, 0, 0, 0)),
            pl.BlockSpec((K, Cout), lambda n, r: (0, 0)),
            pl.BlockSpec((1, Cout), lambda n, r: (0, 0)),
        ],
        out_specs=pl.BlockSpec((1, bh, OW, Cout), lambda n, r: (n, r, 0, 0)),
        compiler_params=pltpu.CompilerParams(**_COMPILER),
    )(x, w, b2)


def _conv7_rgb(x, w, b, act, out_dtype, bh):
    N, H, W, Cin = x.shape
    K, Cout = w.shape
    OW = W
    xp = jnp.pad(x, ((0, 0), (3, 5), (3, 3), (0, 0)))
    Hp = H + 8
    xw = jnp.stack([xp[:, :, j:j + OW, :] for j in range(7)], axis=3)
    xw = xw.reshape(N, Hp, OW * 21)
    b2 = b.astype(jnp.float32).reshape(1, Cout)
    kern = functools.partial(_conv7_rgb_kernel, bh=bh, OW=OW, act=act)
    return pl.pallas_call(
        kern,
        out_shape=jax.ShapeDtypeStruct((N, H, OW, Cout), out_dtype),
        grid=(N, H // bh),
        in_specs=[
            pl.BlockSpec((1, Hp, OW * 21), lambda n, r: (n, 0, 0)),
            pl.BlockSpec((K, Cout), lambda n, r: (0, 0)),
            pl.BlockSpec((1, Cout), lambda n, r: (0, 0)),
        ],
        out_specs=pl.BlockSpec((1, bh, OW, Cout), lambda n, r: (n, r, 0, 0)),
        compiler_params=pltpu.CompilerParams(**_COMPILER),
    )(xw, w, b2)


def _conv_s2(x, w, b, act, out_dtype, bh):
    N, H, W, Cin = x.shape
    K, Cout = w.shape
    OH, OW = H // 2, W // 2
    assert OH % bh == 0, (OH, bh)
    xp = jnp.pad(x, ((0, 0), (1, 1), (1, 1), (0, 0)))
    planes = [xp[:, pi::2, pj::2, :] for pi in (0, 1) for pj in (0, 1)]
    Ph, Pw = OH + 1, OW + 1
    b2 = b.astype(jnp.float32).reshape(1, Cout)
    kern = functools.partial(_conv_s2_kernel, bh=bh, OW=OW, act=act)
    plane_spec = pl.BlockSpec((1, Ph, Pw, Cin), lambda n, r: (n, 0, 0, 0))
    return pl.pallas_call(
        kern,
        out_shape=jax.ShapeDtypeStruct((N, OH, OW, Cout), out_dtype),
        grid=(N, OH // bh),
        in_specs=[plane_spec] * 4 + [
            pl.BlockSpec((K, Cout), lambda n, r: (0, 0)),
            pl.BlockSpec((1, Cout), lambda n, r: (0, 0)),
        ],
        out_specs=pl.BlockSpec((1, bh, OW, Cout), lambda n, r: (n, r, 0, 0)),
        compiler_params=pltpu.CompilerParams(**_COMPILER),
    )(*planes, w, b2)


def _conv_transpose(x, w, b4, out_dtype, bh):
    N, H, W, _ = x.shape
    Cout = w.shape[1] // 4
    xp = jnp.pad(x, ((0, 0), (0, 1), (0, 1), (0, 0)))
    out = _conv_s1(xp, w, b4, 2, 2, (0, 0), "relu", out_dtype, bh)
    out = out.reshape(N, H, W, 2, 2, Cout)
    out = jnp.transpose(out, (0, 1, 3, 2, 4, 5)).reshape(N, 2 * H, 2 * W,
                                                         Cout)
    return out


def kernel(x, w0, b0, w1, b1, w2, b2, w3, b3, w4, b4, w5, b5):
    bf16 = jnp.bfloat16
    h = jnp.transpose(x, (0, 2, 3, 1)).astype(bf16)
    h = _conv7_rgb(h, w0, b0, "relu", bf16, bh=8)
    h = _conv_s2(h, w1, b1, "relu", bf16, bh=16)
    h = _conv_s2(h, w2, b2, "relu", bf16, bh=32)
    h = _conv_transpose(h, w3, b3, bf16, bh=32)
    h = _conv_transpose(h, w4, b4, bf16, bh=16)
    h = _conv_s1(h, w5, b5, 7, 7, (3, 3), "tanh", jnp.float32, bh=8)
    return jnp.transpose(h, (0, 3, 1, 2))
